# baseline (speedup 1.0000x reference)
import functools

import jax
import jax.numpy as jnp
from jax import lax
from jax.experimental import pallas as pl
from jax.experimental.pallas import tpu as pltpu
from jax.experimental.pallas import tpu_sc as plsc

_VOCAB = 30522
_NENT = 100000
_EDIM = 100
_HID = 768
_B = 16
_S = 256
_NTOK = _B * _S
_NW = 32
_TPW = _NTOK // _NW
_ESUP = 400
_EPS = 1e-12


def _split_indices(ids, tts):
    cols = jnp.arange(_S, dtype=jnp.int32)
    wm = (ids > 0) & (ids < _VOCAB)
    wperm = jnp.argsort(jnp.logical_not(wm), axis=1, stable=True).astype(jnp.int32)
    nw = wm.sum(axis=1).astype(jnp.int32)
    wv = cols[None, :] < nw[:, None]
    ids_w = jnp.take_along_axis(ids, wperm, axis=1)
    tts_w = jnp.take_along_axis(tts, wperm, axis=1)
    w_ids = jnp.where(wv, ids_w, 0).astype(jnp.int32)
    w_tt = jnp.where(wv, tts_w, 1).astype(jnp.int32)
    w_pos = jnp.where(wv, wperm, cols[None, :]).astype(jnp.int32)

    km = ids >= _VOCAB
    kperm = jnp.argsort(jnp.logical_not(km), axis=1, stable=True).astype(jnp.int32)
    nk = km.sum(axis=1).astype(jnp.int32)
    kv = (cols[None, :] < nk[:, None]) & (nk[:, None] >= 2)
    ids_k = jnp.take_along_axis(ids, kperm, axis=1)
    tts_k = jnp.take_along_axis(tts, kperm, axis=1)
    k_ent = jnp.where(kv, ids_k - _VOCAB, 0).astype(jnp.int32)
    k_tt = jnp.where(kv, tts_k, 0).astype(jnp.int32)
    k_pos = jnp.where(kv, kperm, 0).astype(jnp.int32)
    k_mask = kv.astype(jnp.float32)
    return w_ids, w_tt, w_pos, k_ent, k_tt, k_pos, k_mask


def _sc_gather(w_ids, w_pos, k_pos, k_sup, word_emb, pos_emb, esup_tab):
    mesh = plsc.VectorSubcoreMesh(core_axis_name="c", subcore_axis_name="s")

    @functools.partial(
        pl.kernel,
        mesh=mesh,
        compiler_params=pltpu.CompilerParams(use_tc_tiling_on_sc=False),
        out_type=[
            jax.ShapeDtypeStruct((_NTOK, _HID), jnp.float32),
            jax.ShapeDtypeStruct((_NTOK, _HID), jnp.float32),
            jax.ShapeDtypeStruct((_NTOK, _HID), jnp.float32),
            jax.ShapeDtypeStruct((_NTOK, _ESUP), jnp.float32),
        ],
        scratch_types=[
            pltpu.VMEM((_TPW,), jnp.int32),
            pltpu.VMEM((_TPW,), jnp.int32),
            pltpu.VMEM((_TPW,), jnp.int32),
            pltpu.VMEM((_TPW // 2,), jnp.int32),
            pltpu.VMEM((_TPW, _HID), jnp.float32),
            pltpu.VMEM((_TPW // 2, _ESUP), jnp.float32),
            pltpu.SemaphoreType.DMA,
        ],
    )
    def gather(wids_h, wpos_h, kpos_h, ksup_h, wemb_h, pemb_h, esup_h,
               W_h, Pw_h, Pk_h, E_h,
               widx_v, pwidx_v, pkidx_v, eidx_v, row_v, erow_v, sem):
        wid = lax.axis_index("s") * 2 + lax.axis_index("c")
        base = wid * _TPW
        half = _TPW // 2
        pltpu.sync_copy(wids_h.at[pl.ds(base, _TPW)], widx_v)
        pltpu.sync_copy(wpos_h.at[pl.ds(base, _TPW)], pwidx_v)
        pltpu.sync_copy(kpos_h.at[pl.ds(base, _TPW)], pkidx_v)
        pltpu.async_copy(wemb_h.at[widx_v], row_v, sem).wait()
        pltpu.sync_copy(row_v, W_h.at[pl.ds(base, _TPW)])
        pltpu.async_copy(pemb_h.at[pwidx_v], row_v, sem).wait()
        pltpu.sync_copy(row_v, Pw_h.at[pl.ds(base, _TPW)])
        pltpu.async_copy(pemb_h.at[pkidx_v], row_v, sem).wait()
        pltpu.sync_copy(row_v, Pk_h.at[pl.ds(base, _TPW)])
        pltpu.sync_copy(ksup_h.at[pl.ds(base, half)], eidx_v)
        pltpu.async_copy(esup_h.at[eidx_v], erow_v, sem).wait()
        pltpu.sync_copy(erow_v, E_h.at[pl.ds(base, half)])
        pltpu.sync_copy(ksup_h.at[pl.ds(base + half, half)], eidx_v)
        pltpu.async_copy(esup_h.at[eidx_v], erow_v, sem).wait()
        pltpu.sync_copy(erow_v, E_h.at[pl.ds(base + half, half)])

    return gather(w_ids, w_pos, k_pos, k_sup, word_emb, pos_emb, esup_tab)


def _tc_body(W, Pw, Pk, Es, wtt, ktt, ksub, kmask,
             tt_emb, wg, wb, keW, keb, kg, kb, out):
    tt0 = tt_emb[0:1, :]
    tt1 = tt_emb[1:2, :]
    wttf = wtt[0].astype(jnp.float32)
    ttrow_w = tt0 * (1.0 - wttf) + tt1 * wttf
    wsum = W[0] + Pw[0] + ttrow_w
    u = jnp.mean(wsum, axis=-1, keepdims=True)
    d = wsum - u
    s = jnp.mean(d * d, axis=-1, keepdims=True)
    wemb = wg[...] * d / jnp.sqrt(s + _EPS) + wb[...]
    km = kmask[0]
    sub = ksub[0]
    es = Es[0]
    e = jnp.where(sub == 0, es[:, 0:100],
                  jnp.where(sub == 1, es[:, 100:200],
                            jnp.where(sub == 2, es[:, 200:300], es[:, 300:400])))
    e = e * km
    proj = lax.dot_general(e, keW[...], (((1,), (1,)), ((), ())),
                           preferred_element_type=jnp.float32)
    kttf = ktt[0].astype(jnp.float32)
    ttrow_k = tt0 * (1.0 - kttf) + tt1 * kttf
    ksum = (proj + keb[...] + ttrow_k + Pk[0]) * km
    uk = jnp.mean(ksum, axis=-1, keepdims=True)
    dk = ksum - uk
    sk = jnp.mean(dk * dk, axis=-1, keepdims=True)
    kemb = kg[...] * dk / jnp.sqrt(sk + _EPS) + kb[...]
    out[0, 0:_S, :] = wemb
    out[0, _S:2 * _S, :] = kemb


def _tc_dense(W, Pw, Pk, Es, wtt, ktt, ksub, kmask, tt_emb, wg, wb, keW, keb, kg, kb):
    b3 = lambda i: (i, 0, 0)
    b2 = lambda i: (0, 0)
    return pl.pallas_call(
        _tc_body,
        grid=(_B,),
        in_specs=[
            pl.BlockSpec((1, _S, _HID), b3),
            pl.BlockSpec((1, _S, _HID), b3),
            pl.BlockSpec((1, _S, _HID), b3),
            pl.BlockSpec((1, _S, _ESUP), b3),
            pl.BlockSpec((1, _S, 1), b3),
            pl.BlockSpec((1, _S, 1), b3),
            pl.BlockSpec((1, _S, 1), b3),
            pl.BlockSpec((1, _S, 1), b3),
            pl.BlockSpec((2, _HID), b2),
            pl.BlockSpec((1, _HID), b2),
            pl.BlockSpec((1, _HID), b2),
            pl.BlockSpec((_HID, _EDIM), b2),
            pl.BlockSpec((1, _HID), b2),
            pl.BlockSpec((1, _HID), b2),
            pl.BlockSpec((1, _HID), b2),
        ],
        out_specs=pl.BlockSpec((1, 2 * _S, _HID), b3),
        out_shape=jax.ShapeDtypeStruct((_B, 2 * _S, _HID), jnp.float32),
    )(W, Pw, Pk, Es, wtt, ktt, ksub, kmask, tt_emb, wg, wb, keW, keb, kg, kb)


def kernel(input_ids, token_type_ids, word_emb, pos_emb, tt_emb, wln_g, wln_b,
           ke_W, ke_b, kln_g, kln_b, entityVec):
    ids = input_ids.astype(jnp.int32)
    tts = token_type_ids.astype(jnp.int32)
    w_ids, w_tt, w_pos, k_ent, k_tt, k_pos, k_mask = _split_indices(ids, tts)

    k_sup = (k_ent // 4).astype(jnp.int32)
    k_sub = (k_ent % 4).astype(jnp.int32)
    esup_tab = entityVec.reshape(_NENT // 4, _ESUP)

    W, Pw, Pk, Es = _sc_gather(
        w_ids.reshape(_NTOK), w_pos.reshape(_NTOK), k_pos.reshape(_NTOK),
        k_sup.reshape(_NTOK), word_emb, pos_emb, esup_tab)

    out = _tc_dense(
        W.reshape(_B, _S, _HID), Pw.reshape(_B, _S, _HID),
        Pk.reshape(_B, _S, _HID), Es.reshape(_B, _S, _ESUP),
        w_tt.reshape(_B, _S, 1), k_tt.reshape(_B, _S, 1),
        k_sub.reshape(_B, _S, 1), k_mask.reshape(_B, _S, 1),
        tt_emb, wln_g.reshape(1, _HID), wln_b.reshape(1, _HID),
        ke_W, ke_b.reshape(1, _HID), kln_g.reshape(1, _HID),
        kln_b.reshape(1, _HID))
    return out


# speedup vs baseline: 1.2646x; 1.2646x over previous
import functools

import jax
import jax.numpy as jnp
from jax import lax
from jax.experimental import pallas as pl
from jax.experimental.pallas import tpu as pltpu
from jax.experimental.pallas import tpu_sc as plsc

_VOCAB = 30522
_NENT = 100000
_EDIM = 100
_HID = 768
_B = 16
_S = 256
_NTOK = _B * _S
_NW = 32
_TPW = _NTOK // _NW
_ESUP = 400
_EPS = 1e-12


def _split_indices(ids, tts):
    cols = jnp.arange(_S, dtype=jnp.int32)
    wm = (ids > 0) & (ids < _VOCAB)
    wperm = jnp.argsort(jnp.logical_not(wm), axis=1, stable=True).astype(jnp.int32)
    nw = wm.sum(axis=1).astype(jnp.int32)
    wv = cols[None, :] < nw[:, None]
    ids_w = jnp.take_along_axis(ids, wperm, axis=1)
    tts_w = jnp.take_along_axis(tts, wperm, axis=1)
    w_ids = jnp.where(wv, ids_w, 0).astype(jnp.int32)
    w_tt = jnp.where(wv, tts_w, 1).astype(jnp.int32)
    w_pos = jnp.where(wv, wperm, cols[None, :]).astype(jnp.int32)

    km = ids >= _VOCAB
    kperm = jnp.argsort(jnp.logical_not(km), axis=1, stable=True).astype(jnp.int32)
    nk = km.sum(axis=1).astype(jnp.int32)
    kv = (cols[None, :] < nk[:, None]) & (nk[:, None] >= 2)
    ids_k = jnp.take_along_axis(ids, kperm, axis=1)
    tts_k = jnp.take_along_axis(tts, kperm, axis=1)
    k_ent = jnp.where(kv, ids_k - _VOCAB, 0).astype(jnp.int32)
    k_tt = jnp.where(kv, tts_k, 0).astype(jnp.int32)
    k_pos = jnp.where(kv, kperm, 0).astype(jnp.int32)
    k_mask = kv.astype(jnp.float32)
    return w_ids, w_tt, w_pos, k_ent, k_tt, k_pos, k_mask


def _sc_gather(w_ids, w_pos, k_pos, k_ent, word_emb, pos_emb, ent128):
    mesh = plsc.VectorSubcoreMesh(core_axis_name="c", subcore_axis_name="s")
    half = _TPW // 2

    @functools.partial(
        pl.kernel,
        mesh=mesh,
        out_type=[
            jax.ShapeDtypeStruct((_NTOK, _HID), jnp.float32),
            jax.ShapeDtypeStruct((_NTOK, _HID), jnp.float32),
            jax.ShapeDtypeStruct((_NTOK, _HID), jnp.float32),
            jax.ShapeDtypeStruct((_NTOK, 128), jnp.float32),
        ],
        scratch_types=[
            pltpu.VMEM((_TPW,), jnp.int32),
            pltpu.VMEM((_TPW,), jnp.int32),
            pltpu.VMEM((_TPW,), jnp.int32),
            pltpu.VMEM((_TPW,), jnp.int32),
            pltpu.VMEM((half, _HID), jnp.float32),
            pltpu.VMEM((half, _HID), jnp.float32),
            pltpu.VMEM((_TPW, 128), jnp.float32),
            pltpu.SemaphoreType.DMA,
            pltpu.SemaphoreType.DMA,
            pltpu.SemaphoreType.DMA,
            pltpu.SemaphoreType.DMA,
        ],
    )
    def gather(wids_h, wpos_h, kpos_h, kent_h, wemb_h, pemb_h, ent_h,
               W_h, Pw_h, Pk_h, E_h,
               widx_v, pwidx_v, pkidx_v, keidx_v, bufA, bufB, ebuf,
               gsem, esem, wsemA, wsemB):
        wid = lax.axis_index("s") * 2 + lax.axis_index("c")
        base = wid * _TPW
        pltpu.sync_copy(wids_h.at[pl.ds(base, _TPW)], widx_v)
        pltpu.sync_copy(wpos_h.at[pl.ds(base, _TPW)], pwidx_v)
        pltpu.sync_copy(kpos_h.at[pl.ds(base, _TPW)], pkidx_v)
        pltpu.sync_copy(kent_h.at[pl.ds(base, _TPW)], keidx_v)
        eg = pltpu.async_copy(ent_h.at[keidx_v], ebuf, esem)

        stages = [
            (wemb_h, widx_v, 0, W_h), (wemb_h, widx_v, half, W_h),
            (pemb_h, pwidx_v, 0, Pw_h), (pemb_h, pwidx_v, half, Pw_h),
            (pemb_h, pkidx_v, 0, Pk_h), (pemb_h, pkidx_v, half, Pk_h),
        ]
        bufs = (bufA, bufB)
        wsems = (wsemA, wsemB)
        writes = [None, None]
        for i, (tab, idx, off, dst) in enumerate(stages):
            buf = bufs[i % 2]
            if writes[i % 2] is not None:
                writes[i % 2].wait()
            pltpu.async_copy(tab.at[idx.at[pl.ds(off, half)]], buf, gsem).wait()
            writes[i % 2] = pltpu.async_copy(
                buf, dst.at[pl.ds(base + off, half)], wsems[i % 2])
        eg.wait()
        ew = pltpu.async_copy(ebuf, E_h.at[pl.ds(base, _TPW)], esem)
        writes[0].wait()
        writes[1].wait()
        ew.wait()

    return gather(w_ids, w_pos, k_pos, k_ent, word_emb, pos_emb, ent128)


def _tc_body(W, Pw, Pk, Es, wtt, ktt, kmask,
             tt_emb, wg, wb, keW, keb, kg, kb, out):
    tt0 = tt_emb[0:1, :]
    tt1 = tt_emb[1:2, :]
    wttf = wtt[0].astype(jnp.float32)
    ttrow_w = tt0 * (1.0 - wttf) + tt1 * wttf
    wsum = W[0] + Pw[0] + ttrow_w
    u = jnp.mean(wsum, axis=-1, keepdims=True)
    d = wsum - u
    s = jnp.mean(d * d, axis=-1, keepdims=True)
    wemb = wg[...] * d / jnp.sqrt(s + _EPS) + wb[...]
    km = kmask[0]
    e = Es[0][:, 0:_EDIM] * km
    proj = lax.dot_general(e, keW[...], (((1,), (1,)), ((), ())),
                           preferred_element_type=jnp.float32)
    kttf = ktt[0].astype(jnp.float32)
    ttrow_k = tt0 * (1.0 - kttf) + tt1 * kttf
    ksum = (proj + keb[...] + ttrow_k + Pk[0]) * km
    uk = jnp.mean(ksum, axis=-1, keepdims=True)
    dk = ksum - uk
    sk = jnp.mean(dk * dk, axis=-1, keepdims=True)
    kemb = kg[...] * dk / jnp.sqrt(sk + _EPS) + kb[...]
    out[0, 0:_S, :] = wemb
    out[0, _S:2 * _S, :] = kemb


def _tc_dense(W, Pw, Pk, Es, wtt, ktt, kmask, tt_emb, wg, wb, keW, keb, kg, kb):
    b3 = lambda i: (i, 0, 0)
    b2 = lambda i: (0, 0)
    return pl.pallas_call(
        _tc_body,
        grid=(_B,),
        in_specs=[
            pl.BlockSpec((1, _S, _HID), b3),
            pl.BlockSpec((1, _S, _HID), b3),
            pl.BlockSpec((1, _S, _HID), b3),
            pl.BlockSpec((1, _S, 128), b3),
            pl.BlockSpec((1, _S, 1), b3),
            pl.BlockSpec((1, _S, 1), b3),
            pl.BlockSpec((1, _S, 1), b3),
            pl.BlockSpec((2, _HID), b2),
            pl.BlockSpec((1, _HID), b2),
            pl.BlockSpec((1, _HID), b2),
            pl.BlockSpec((_HID, _EDIM), b2),
            pl.BlockSpec((1, _HID), b2),
            pl.BlockSpec((1, _HID), b2),
            pl.BlockSpec((1, _HID), b2),
        ],
        out_specs=pl.BlockSpec((1, 2 * _S, _HID), b3),
        out_shape=jax.ShapeDtypeStruct((_B, 2 * _S, _HID), jnp.float32),
    )(W, Pw, Pk, Es, wtt, ktt, kmask, tt_emb, wg, wb, keW, keb, kg, kb)


def kernel(input_ids, token_type_ids, word_emb, pos_emb, tt_emb, wln_g, wln_b,
           ke_W, ke_b, kln_g, kln_b, entityVec):
    ids = input_ids.astype(jnp.int32)
    tts = token_type_ids.astype(jnp.int32)
    w_ids, w_tt, w_pos, k_ent, k_tt, k_pos, k_mask = _split_indices(ids, tts)

    ent128 = jnp.pad(entityVec, ((0, 0), (0, 128 - _EDIM)))

    W, Pw, Pk, Es = _sc_gather(
        w_ids.reshape(_NTOK), w_pos.reshape(_NTOK), k_pos.reshape(_NTOK),
        k_ent.reshape(_NTOK), word_emb, pos_emb, ent128)

    out = _tc_dense(
        W.reshape(_B, _S, _HID), Pw.reshape(_B, _S, _HID),
        Pk.reshape(_B, _S, _HID), Es.reshape(_B, _S, 128),
        w_tt.reshape(_B, _S, 1), k_tt.reshape(_B, _S, 1),
        k_mask.reshape(_B, _S, 1),
        tt_emb, wln_g.reshape(1, _HID), wln_b.reshape(1, _HID),
        ke_W, ke_b.reshape(1, _HID), kln_g.reshape(1, _HID),
        kln_b.reshape(1, _HID))
    return out


# speedup vs baseline: 2.0219x; 1.5989x over previous
import functools

import jax
import jax.numpy as jnp
from jax import lax
from jax.experimental import pallas as pl
from jax.experimental.pallas import tpu as pltpu
from jax.experimental.pallas import tpu_sc as plsc

_VOCAB = 30522
_NENT = 100000
_EDIM = 100
_HID = 768
_MAXP = 512
_B = 16
_S = 256
_NTOK = _B * _S
_NW = 32
_TPW = _NTOK // _NW
_EPS = 1e-12
_PT = _MAXP + 2


def _split_indices(ids, tts):
    cols = jnp.arange(_S, dtype=jnp.int32)
    wm = (ids > 0) & (ids < _VOCAB)
    wperm = jnp.argsort(jnp.logical_not(wm), axis=1, stable=True).astype(jnp.int32)
    nw = wm.sum(axis=1).astype(jnp.int32)
    wv = cols[None, :] < nw[:, None]
    ids_w = jnp.take_along_axis(ids, wperm, axis=1)
    tts_w = jnp.take_along_axis(tts, wperm, axis=1)
    w_ids = jnp.where(wv, ids_w, 0).astype(jnp.int32)
    w_tt = jnp.where(wv, tts_w, 1).astype(jnp.int32)
    w_pos = jnp.where(wv, wperm, cols[None, :]).astype(jnp.int32)

    km = ids >= _VOCAB
    kperm = jnp.argsort(jnp.logical_not(km), axis=1, stable=True).astype(jnp.int32)
    nk = km.sum(axis=1).astype(jnp.int32)
    kv = (cols[None, :] < nk[:, None]) & (nk[:, None] >= 2)
    ids_k = jnp.take_along_axis(ids, kperm, axis=1)
    tts_k = jnp.take_along_axis(tts, kperm, axis=1)
    k_ent = jnp.where(kv, ids_k - _VOCAB, 0).astype(jnp.int32)
    k_tt = jnp.where(kv, tts_k, 0).astype(jnp.int32)
    k_pos = jnp.where(kv, kperm, 0).astype(jnp.int32)
    k_mask = kv.astype(jnp.float32)
    return w_ids, w_tt, w_pos, k_ent, k_tt, k_pos, k_mask


_EBLK = 2048


def _relayout_body(entT, eye, out):
    x = entT[...]
    xt = lax.dot_general(x, eye[...], (((0,), (0,)), ((), ())),
                         preferred_element_type=jnp.float32)
    out[...] = jnp.concatenate(
        [xt, jnp.zeros((_EBLK, 128 - _EDIM), jnp.float32)], axis=1)


def _tc_relayout(entT, eye):
    return pl.pallas_call(
        _relayout_body,
        grid=((_NENT + _EBLK - 1) // _EBLK,),
        in_specs=[
            pl.BlockSpec((_EDIM, _EBLK), lambda i: (0, i)),
            pl.BlockSpec((_EDIM, _EDIM), lambda i: (0, 0)),
        ],
        out_specs=pl.BlockSpec((_EBLK, 128), lambda i: (i, 0)),
        out_shape=jax.ShapeDtypeStruct((_NENT, 128), jnp.float32),
    )(entT, eye)


def _sc_gather_word(w_ids, word_emb):
    mesh = plsc.VectorSubcoreMesh(core_axis_name="c", subcore_axis_name="s")
    half = _TPW // 2

    @functools.partial(
        pl.kernel,
        mesh=mesh,
        out_type=jax.ShapeDtypeStruct((_NTOK, _HID), jnp.float32),
        scratch_types=[
            pltpu.VMEM((_TPW,), jnp.int32),
            pltpu.VMEM((half, _HID), jnp.float32),
            pltpu.VMEM((half, _HID), jnp.float32),
            pltpu.SemaphoreType.DMA,
            pltpu.SemaphoreType.DMA,
            pltpu.SemaphoreType.DMA,
        ],
    )
    def gather(wids_h, wemb_h, W_h, widx_v, bufA, bufB, gsem, wsemA, wsemB):
        wid = lax.axis_index("s") * 2 + lax.axis_index("c")
        base = wid * _TPW
        pltpu.sync_copy(wids_h.at[pl.ds(base, _TPW)], widx_v)
        g0 = pltpu.async_copy(wemb_h.at[widx_v.at[pl.ds(0, half)]], bufA, gsem)
        g1 = pltpu.async_copy(wemb_h.at[widx_v.at[pl.ds(half, half)]], bufB, gsem)
        g0.wait()
        w0 = pltpu.async_copy(bufA, W_h.at[pl.ds(base, half)], wsemA)
        g1.wait()
        w1 = pltpu.async_copy(bufB, W_h.at[pl.ds(base + half, half)], wsemB)
        w0.wait()
        w1.wait()

    return gather(w_ids, word_emb)


def _sc_gather_ent(k_ent, ent128):
    mesh = plsc.VectorSubcoreMesh(core_axis_name="c", subcore_axis_name="s")

    @functools.partial(
        pl.kernel,
        mesh=mesh,
        out_type=jax.ShapeDtypeStruct((_NTOK, 128), jnp.float32),
        scratch_types=[
            pltpu.VMEM((_TPW,), jnp.int32),
            pltpu.VMEM((_TPW, 128), jnp.float32),
            pltpu.SemaphoreType.DMA,
        ],
    )
    def gather(kent_h, ent_h, E_h, keidx_v, ebuf, sem):
        wid = lax.axis_index("s") * 2 + lax.axis_index("c")
        base = wid * _TPW
        pltpu.sync_copy(kent_h.at[pl.ds(base, _TPW)], keidx_v)
        pltpu.async_copy(ent_h.at[keidx_v], ebuf, sem).wait()
        pltpu.sync_copy(ebuf, E_h.at[pl.ds(base, _TPW)])

    return gather(k_ent, ent128)


def _tc_body(W, Es, wtt, ktt, wpos, kpos, kmask,
             ptab, wg, wb, keW, keb, kg, kb, out):
    cols = lax.broadcasted_iota(jnp.int32, (_S, _PT), 1)
    pt = ptab[...]
    oh_w = ((cols == wpos[0]) | (cols == wtt[0] + _MAXP)).astype(jnp.float32)
    wsum = W[0] + lax.dot_general(oh_w, pt, (((1,), (0,)), ((), ())),
                                  preferred_element_type=jnp.float32)
    u = jnp.mean(wsum, axis=-1, keepdims=True)
    d = wsum - u
    s = jnp.mean(d * d, axis=-1, keepdims=True)
    wemb = wg[...] * d / jnp.sqrt(s + _EPS) + wb[...]
    km = kmask[0]
    proj = lax.dot_general(Es[0][:, 0:_EDIM], keW[...], (((1,), (1,)), ((), ())),
                           preferred_element_type=jnp.float32)
    oh_k = ((cols == kpos[0]) | (cols == ktt[0] + _MAXP)).astype(jnp.float32)
    ptk = lax.dot_general(oh_k, pt, (((1,), (0,)), ((), ())),
                          preferred_element_type=jnp.float32)
    ksum = (proj + keb[...] + ptk) * km
    uk = jnp.mean(ksum, axis=-1, keepdims=True)
    dk = ksum - uk
    sk = jnp.mean(dk * dk, axis=-1, keepdims=True)
    kemb = kg[...] * dk / jnp.sqrt(sk + _EPS) + kb[...]
    out[0, 0:_S, :] = wemb
    out[0, _S:2 * _S, :] = kemb


def _tc_dense(W, Es, wtt, ktt, wpos, kpos, kmask, ptab, wg, wb, keW, keb, kg, kb):
    b3 = lambda i: (i, 0, 0)
    b2 = lambda i: (0, 0)
    return pl.pallas_call(
        _tc_body,
        grid=(_B,),
        in_specs=[
            pl.BlockSpec((1, _S, _HID), b3),
            pl.BlockSpec((1, _S, 128), b3),
            pl.BlockSpec((1, _S, 1), b3),
            pl.BlockSpec((1, _S, 1), b3),
            pl.BlockSpec((1, _S, 1), b3),
            pl.BlockSpec((1, _S, 1), b3),
            pl.BlockSpec((1, _S, 1), b3),
            pl.BlockSpec((_PT, _HID), b2),
            pl.BlockSpec((1, _HID), b2),
            pl.BlockSpec((1, _HID), b2),
            pl.BlockSpec((_HID, _EDIM), b2),
            pl.BlockSpec((1, _HID), b2),
            pl.BlockSpec((1, _HID), b2),
            pl.BlockSpec((1, _HID), b2),
        ],
        out_specs=pl.BlockSpec((1, 2 * _S, _HID), b3),
        out_shape=jax.ShapeDtypeStruct((_B, 2 * _S, _HID), jnp.float32),
    )(W, Es, wtt, ktt, wpos, kpos, kmask, ptab, wg, wb, keW, keb, kg, kb)


def kernel(input_ids, token_type_ids, word_emb, pos_emb, tt_emb, wln_g, wln_b,
           ke_W, ke_b, kln_g, kln_b, entityVec):
    ids = input_ids.astype(jnp.int32)
    tts = token_type_ids.astype(jnp.int32)
    w_ids, w_tt, w_pos, k_ent, k_tt, k_pos, k_mask = _split_indices(ids, tts)

    entT = jnp.transpose(entityVec)
    eye = jnp.eye(_EDIM, dtype=jnp.float32)
    ent128 = _tc_relayout(entT, eye)

    W = _sc_gather_word(w_ids.reshape(_NTOK), word_emb)
    Es = _sc_gather_ent(k_ent.reshape(_NTOK), ent128)

    ptab = jnp.concatenate([pos_emb, tt_emb], axis=0)

    out = _tc_dense(
        W.reshape(_B, _S, _HID), Es.reshape(_B, _S, 128),
        w_tt.reshape(_B, _S, 1), k_tt.reshape(_B, _S, 1),
        w_pos.reshape(_B, _S, 1), k_pos.reshape(_B, _S, 1),
        k_mask.reshape(_B, _S, 1),
        ptab, wln_g.reshape(1, _HID), wln_b.reshape(1, _HID),
        ke_W, ke_b.reshape(1, _HID), kln_g.reshape(1, _HID),
        kln_b.reshape(1, _HID))
    return out
